# baseline (speedup 1.0000x reference)
import jax
import jax.numpy as jnp
from jax import lax
from jax.experimental import pallas as pl
from jax.experimental.pallas import tpu as pltpu
from jax.experimental.pallas import tpu_sc as plsc

_C, _G, _S, _L = 16, 1024, 32, 2
_B = 64
_M = 4
_STEPS = 3
_GAMMA = 0.01
_INV_GAMMA = 1.0 / _GAMMA
_CG = _C * _G
_NTILES = 32
_PER_TILE = _CG // _NTILES
_CHUNK = 128
_NCHUNK = _PER_TILE // _CHUNK
_NBG = _B // 16
_SHIFT = 52.0


def _sc_stage(xt10, idx):
    info = plsc.get_sparse_core_info()
    nc = info.num_cores
    mesh = plsc.VectorSubcoreMesh(core_axis_name="c", subcore_axis_name="s")

    idx_words = _PER_TILE * _L * _S

    def body(x_hbm, idx_hbm, se_hbm, x_v, idx_v, se_st):
        wid = lax.axis_index("s") * nc + lax.axis_index("c")
        pltpu.sync_copy(x_hbm, x_v)
        pltpu.sync_copy(idx_hbm.at[wid], idx_v)

        for chunk in range(_NCHUNK):
            @plsc.parallel_loop(0, _CHUNK, unroll=4)
            def cg_body(j):
                pbase = (chunk * _CHUNK + j) * (_L * _S)
                iv0 = [idx_v[pl.ds(pbase + h * 16, 16)] * 64
                       for h in range(2)]
                iv1 = [idx_v[pl.ds(pbase + 32 + h * 16, 16)] * 64
                       for h in range(2)]
                sm = [jnp.zeros((16,), jnp.float32) for _ in range(_NBG)]
                for s in range(_S):
                    h, k = divmod(s, 16)
                    a0 = iv0[h][k]
                    a1 = iv1[h][k]
                    for g in range(_NBG):
                        v0 = x_v[pl.ds(a0 + g * 16, 16)]
                        v1 = x_v[pl.ds(a1 + g * 16, 16)]
                        sm[g] = sm[g] + jnp.exp(v0 * v1 - _SHIFT)
                for g in range(_NBG):
                    se_st[pl.ds(j * _B + g * 16, 16)] = sm[g]

            out_slice = pl.ds((wid * _PER_TILE + chunk * _CHUNK) * _B,
                              _CHUNK * _B)
            pltpu.sync_copy(se_st, se_hbm.at[out_slice])

    f = pl.kernel(
        body,
        out_type=jax.ShapeDtypeStruct((_CG * _B,), jnp.float32),
        mesh=mesh,
        compiler_params=pltpu.CompilerParams(needs_layout_passes=False),
        scratch_types=[
            pltpu.VMEM((_G * _B,), jnp.float32),
            pltpu.VMEM((idx_words,), jnp.int32),
            pltpu.VMEM((_CHUNK * _B,), jnp.float32),
        ],
    )
    return f(xt10.reshape(_G * _B), idx.reshape(_NTILES, idx_words))


def _norm(lse):
    m = jnp.max(lse)
    return jnp.where(m > 1.0, lse / m, lse)


_ROWS = _G * _B // 128


def _tc_combine(se, W, rt):
    def body(se_ref, w_ref, rt_ref, out_ref, out10_ref):
        w = w_ref[...]
        wmx = jnp.max(w, axis=1, keepdims=True)
        we = jnp.exp(w - wmx)
        ws = we / jnp.sum(we, axis=1, keepdims=True)

        hs = [jnp.zeros((_ROWS, 128), jnp.float32) for _ in range(_M)]
        for c in range(_C):
            lse = _GAMMA * jnp.log(se_ref[c]) + (_GAMMA * _SHIFT)
            cl = _norm(lse)
            for m in range(_M):
                hs[m] = hs[m] + cl * ws[m, c]
        hmx = jnp.maximum(jnp.maximum(hs[0], hs[1]),
                          jnp.maximum(hs[2], hs[3]))
        hse = sum(jnp.exp((h - hmx) * _INV_GAMMA) for h in hs)
        r = _norm(_GAMMA * jnp.log(hse) + hmx)

        R = rt_ref[...]
        pmx = jnp.maximum(R, r)
        pse = jnp.exp((R - pmx) * _INV_GAMMA) + jnp.exp((r - pmx) * _INV_GAMMA)
        rn = _norm(_GAMMA * jnp.log(pse) + pmx)
        out_ref[...] = rn
        out10_ref[...] = rn * 10.0

    return pl.pallas_call(
        body,
        out_shape=(jax.ShapeDtypeStruct((_ROWS, 128), jnp.float32),
                   jax.ShapeDtypeStruct((_ROWS, 128), jnp.float32)),
    )(se, W, rt)


def kernel(x, W, I):
    idx = jnp.transpose(I, (0, 1, 3, 2)).reshape(_CG, _L, _S).astype(jnp.int32)
    rt = jnp.transpose(x).reshape(_ROWS, 128)
    rt10 = rt * 10.0
    for _ in range(_STEPS):
        se = _sc_stage(rt10, idx)
        rt, rt10 = _tc_combine(se.reshape(_C, _ROWS, 128), W, rt)
    return jnp.transpose(rt.reshape(_G, _B))


# speedup vs baseline: 1.1419x; 1.1419x over previous
import jax
import jax.numpy as jnp
from jax import lax
from jax.experimental import pallas as pl
from jax.experimental.pallas import tpu as pltpu
from jax.experimental.pallas import tpu_sc as plsc

_C, _G, _S, _L = 16, 1024, 32, 2
_B = 64
_M = 4
_STEPS = 3
_GAMMA = 0.01
_INV_GAMMA = 1.0 / _GAMMA
_CG = _C * _G
_NTILES = 32
_PER_TILE = _CG // _NTILES
_CHUNK = 256
_NCHUNK = _PER_TILE // _CHUNK
_NBG = _B // 16
_SHIFT = 52.0


def _sc_stage(xt10, idx):
    info = plsc.get_sparse_core_info()
    nc = info.num_cores
    mesh = plsc.VectorSubcoreMesh(core_axis_name="c", subcore_axis_name="s")

    idx_words = _PER_TILE * _L * _S

    def body(x_hbm, idx_hbm, se_hbm, x_v, idx_v, se_st):
        wid = lax.axis_index("s") * nc + lax.axis_index("c")
        pltpu.sync_copy(x_hbm, x_v)
        pltpu.sync_copy(idx_hbm.at[wid], idx_v)

        for chunk in range(_NCHUNK):
            @plsc.parallel_loop(0, _CHUNK, unroll=2)
            def cg_body(j):
                pbase = (chunk * _CHUNK + j) * (_L * _S)
                iv0 = [idx_v[pl.ds(pbase + h * 16, 16)] * 64
                       for h in range(2)]
                iv1 = [idx_v[pl.ds(pbase + 32 + h * 16, 16)] * 64
                       for h in range(2)]
                sm = [jnp.zeros((16,), jnp.float32) for _ in range(_NBG)]
                for s in range(_S):
                    h, k = divmod(s, 16)
                    a0 = iv0[h][k]
                    a1 = iv1[h][k]
                    for g in range(_NBG):
                        v0 = x_v[pl.ds(a0 + g * 16, 16)]
                        v1 = x_v[pl.ds(a1 + g * 16, 16)]
                        sm[g] = sm[g] + jnp.exp(v0 * v1 - _SHIFT)
                for g in range(_NBG):
                    se_st[pl.ds(j * _B + g * 16, 16)] = sm[g]

            out_slice = pl.ds((wid * _PER_TILE + chunk * _CHUNK) * _B,
                              _CHUNK * _B)
            pltpu.sync_copy(se_st, se_hbm.at[out_slice])

    f = pl.kernel(
        body,
        out_type=jax.ShapeDtypeStruct((_CG * _B,), jnp.float32),
        mesh=mesh,
        compiler_params=pltpu.CompilerParams(needs_layout_passes=False),
        scratch_types=[
            pltpu.VMEM((_G * _B,), jnp.float32),
            pltpu.VMEM((idx_words,), jnp.int32),
            pltpu.VMEM((_CHUNK * _B,), jnp.float32),
        ],
    )
    return f(xt10.reshape(_G * _B), idx.reshape(_NTILES, idx_words))


def _norm(lse):
    m = jnp.max(lse)
    return jnp.where(m > 1.0, lse / m, lse)


_ROWS = _G * _B // 128


def _tc_combine(se, W, rt):
    def body(se_ref, w_ref, rt_ref, out_ref, out10_ref):
        w = w_ref[...]
        wmx = jnp.max(w, axis=1, keepdims=True)
        we = jnp.exp(w - wmx)
        ws = we / jnp.sum(we, axis=1, keepdims=True)

        hs = [jnp.zeros((_ROWS, 128), jnp.float32) for _ in range(_M)]
        for c in range(_C):
            lse = _GAMMA * jnp.log(se_ref[c]) + (_GAMMA * _SHIFT)
            cl = _norm(lse)
            for m in range(_M):
                hs[m] = hs[m] + cl * ws[m, c]
        hmx = jnp.maximum(jnp.maximum(hs[0], hs[1]),
                          jnp.maximum(hs[2], hs[3]))
        hse = sum(jnp.exp((h - hmx) * _INV_GAMMA) for h in hs)
        r = _norm(_GAMMA * jnp.log(hse) + hmx)

        R = rt_ref[...]
        pmx = jnp.maximum(R, r)
        pse = jnp.exp((R - pmx) * _INV_GAMMA) + jnp.exp((r - pmx) * _INV_GAMMA)
        rn = _norm(_GAMMA * jnp.log(pse) + pmx)
        out_ref[...] = rn
        out10_ref[...] = rn * 10.0

    return pl.pallas_call(
        body,
        out_shape=(jax.ShapeDtypeStruct((_ROWS, 128), jnp.float32),
                   jax.ShapeDtypeStruct((_ROWS, 128), jnp.float32)),
    )(se, W, rt)


def kernel(x, W, I):
    idx = jnp.transpose(I, (0, 1, 3, 2)).reshape(_CG, _L, _S).astype(jnp.int32)
    rt = jnp.transpose(x).reshape(_ROWS, 128)
    rt10 = rt * 10.0
    for _ in range(_STEPS):
        se = _sc_stage(rt10, idx)
        rt, rt10 = _tc_combine(se.reshape(_C, _ROWS, 128), W, rt)
    return jnp.transpose(rt.reshape(_G, _B))


# speedup vs baseline: 1.1523x; 1.0091x over previous
import jax
import jax.numpy as jnp
from jax import lax
from jax.experimental import pallas as pl
from jax.experimental.pallas import tpu as pltpu
from jax.experimental.pallas import tpu_sc as plsc

_C, _G, _S, _L = 16, 1024, 32, 2
_B = 64
_M = 4
_STEPS = 3
_GAMMA = 0.01
_INV_GAMMA = 1.0 / _GAMMA
_CG = _C * _G
_NTILES = 32
_PER_TILE = _CG // _NTILES
_CHUNK = 256
_NCHUNK = _PER_TILE // _CHUNK
_NBG = _B // 16
_SHIFT = 52.0


def _sc_stage(xt10, idx):
    info = plsc.get_sparse_core_info()
    nc = info.num_cores
    mesh = plsc.VectorSubcoreMesh(core_axis_name="c", subcore_axis_name="s")

    idx_words = _PER_TILE * _L * _S

    def body(x_hbm, idx_hbm, se_hbm, x_v, idx_v, se_st, sem_x, sem_i):
        wid = lax.axis_index("s") * nc + lax.axis_index("c")
        cp_x = pltpu.async_copy(x_hbm, x_v, sem_x)
        cp_i = pltpu.async_copy(idx_hbm.at[wid], idx_v, sem_i)
        cp_x.wait()
        cp_i.wait()

        for chunk in range(_NCHUNK):
            @plsc.parallel_loop(0, _CHUNK, unroll=2)
            def cg_body(j):
                pbase = (chunk * _CHUNK + j) * (_L * _S)
                iv0 = [idx_v[pl.ds(pbase + h * 16, 16)] * 64
                       for h in range(2)]
                iv1 = [idx_v[pl.ds(pbase + 32 + h * 16, 16)] * 64
                       for h in range(2)]
                sm = [jnp.zeros((16,), jnp.float32) for _ in range(_NBG)]
                for s in range(_S):
                    h, k = divmod(s, 16)
                    a0 = iv0[h][k]
                    a1 = iv1[h][k]
                    for g in range(_NBG):
                        v0 = x_v[pl.ds(a0 + g * 16, 16)]
                        v1 = x_v[pl.ds(a1 + g * 16, 16)]
                        sm[g] = sm[g] + jnp.exp(v0 * v1 - _SHIFT)
                for g in range(_NBG):
                    se_st[pl.ds(j * _B + g * 16, 16)] = sm[g]

            out_slice = pl.ds((wid * _PER_TILE + chunk * _CHUNK) * _B,
                              _CHUNK * _B)
            pltpu.sync_copy(se_st, se_hbm.at[out_slice])

    f = pl.kernel(
        body,
        out_type=jax.ShapeDtypeStruct((_CG * _B,), jnp.float32),
        mesh=mesh,
        compiler_params=pltpu.CompilerParams(needs_layout_passes=False),
        scratch_types=[
            pltpu.VMEM((_G * _B,), jnp.float32),
            pltpu.VMEM((idx_words,), jnp.int32),
            pltpu.VMEM((_CHUNK * _B,), jnp.float32),
            pltpu.SemaphoreType.DMA,
            pltpu.SemaphoreType.DMA,
        ],
    )
    return f(xt10.reshape(_G * _B), idx.reshape(_NTILES, idx_words))


def _norm(lse):
    m = jnp.max(lse)
    return jnp.where(m > 1.0, lse / m, lse)


_ROWS = _G * _B // 128


def _tc_combine(se, W, rt):
    def body(se_ref, w_ref, rt_ref, out_ref, out10_ref):
        w = w_ref[...]
        wmx = jnp.max(w, axis=1, keepdims=True)
        we = jnp.exp(w - wmx)
        ws = we / jnp.sum(we, axis=1, keepdims=True)

        hs = [jnp.zeros((_ROWS, 128), jnp.float32) for _ in range(_M)]
        for c in range(_C):
            lse = _GAMMA * jnp.log(se_ref[c]) + (_GAMMA * _SHIFT)
            cl = _norm(lse)
            for m in range(_M):
                hs[m] = hs[m] + cl * ws[m, c]
        hmx = jnp.maximum(jnp.maximum(hs[0], hs[1]),
                          jnp.maximum(hs[2], hs[3]))
        hse = sum(jnp.exp((h - hmx) * _INV_GAMMA) for h in hs)
        r = _norm(_GAMMA * jnp.log(hse) + hmx)

        R = rt_ref[...]
        pmx = jnp.maximum(R, r)
        pse = jnp.exp((R - pmx) * _INV_GAMMA) + jnp.exp((r - pmx) * _INV_GAMMA)
        rn = _norm(_GAMMA * jnp.log(pse) + pmx)
        out_ref[...] = rn
        out10_ref[...] = rn * 10.0

    return pl.pallas_call(
        body,
        out_shape=(jax.ShapeDtypeStruct((_ROWS, 128), jnp.float32),
                   jax.ShapeDtypeStruct((_ROWS, 128), jnp.float32)),
    )(se, W, rt)


def kernel(x, W, I):
    idx = jnp.transpose(I, (0, 1, 3, 2)).reshape(_CG, _L, _S).astype(jnp.int32)
    rt = jnp.transpose(x).reshape(_ROWS, 128)
    rt10 = rt * 10.0
    for _ in range(_STEPS):
        se = _sc_stage(rt10, idx)
        rt, rt10 = _tc_combine(se.reshape(_C, _ROWS, 128), W, rt)
    return jnp.transpose(rt.reshape(_G, _B))
